# baseline (speedup 1.0000x reference)
import functools

import jax
import jax.numpy as jnp
from jax import lax
from jax.experimental import pallas as pl
from jax.experimental.pallas import tpu as pltpu
from jax.experimental.pallas import tpu_sc as plsc

VOCAB = 100000
D = 64
N_OUT = 128
B = 4096
L = 50

NC, NS = 2, 16
NW = NC * NS
BPW = B // NW
IPS = 128
NSTEPS = BPW * L // IPS
RING = 8
RROWS = RING * IPS

_mesh = plsc.VectorSubcoreMesh(core_axis_name="c", subcore_axis_name="s")


@functools.partial(
    pl.kernel,
    mesh=_mesh,
    compiler_params=pltpu.CompilerParams(use_tc_tiling_on_sc=False),
    out_type=jax.ShapeDtypeStruct((B, D), jnp.float32),
    scratch_types=[
        pltpu.VMEM((NSTEPS, IPS), jnp.int32),
        pltpu.VMEM((RROWS, D), jnp.float32),
        pltpu.VMEM((BPW, D), jnp.float32),
        [pltpu.SemaphoreType.DMA] * RING,
    ],
)
def _sc_pool(idx_hbm, table_hbm, out_hbm, idx_v, rows_v, pool_v, sems):
    wid = lax.axis_index("s") * NC + lax.axis_index("c")

    pltpu.sync_copy(idx_hbm.at[pl.ds(wid * NSTEPS, NSTEPS)], idx_v)

    def gather(r, slot):
        return pltpu.make_async_copy(
            table_hbm.at[idx_v.at[r]],
            rows_v.at[pl.ds(slot * IPS, IPS)],
            sems[slot],
        )

    for slot in range(RING - 1):
        gather(slot, slot).start()

    def pool_bag(c):
        base = L * c

        def ld(l, off):
            rr = jnp.bitwise_and(base + l, RROWS - 1)
            return rows_v[rr, pl.ds(off, 16)]

        def body(l, acc):
            return (
                jnp.maximum(acc[0], ld(l, 0)),
                jnp.maximum(acc[1], ld(l, 16)),
                jnp.maximum(acc[2], ld(l, 32)),
                jnp.maximum(acc[3], ld(l, 48)),
            )

        a0, a1, a2, a3 = lax.fori_loop(
            1, L, body, (ld(0, 0), ld(0, 16), ld(0, 32), ld(0, 48)),
            unroll=7,
        )
        pool_v[c, pl.ds(0, 16)] = a0
        pool_v[c, pl.ds(16, 16)] = a1
        pool_v[c, pl.ds(32, 16)] = a2
        pool_v[c, pl.ds(48, 16)] = a3

    def step_work(r, slot, c):
        gather(r, slot).wait()

        pool_bag(c)
        pool_bag(c + 1)
        third = L * (c + 2) + L <= IPS * (r + 1)

        @pl.when(third)
        def _():
            pool_bag(c + 2)

        c = c + jnp.where(third, jnp.int32(3), jnp.int32(2))

        @pl.when(r + RING - 1 < NSTEPS)
        def _():
            gather(r + RING - 1, (slot + RING - 1) % RING).start()

        return c

    def outer(k, c):
        for b in range(RING):
            c = step_work(RING * k + b, b, c)
        return c

    c = lax.fori_loop(0, NSTEPS // RING, outer, jnp.int32(0))
    for r in range(RING * (NSTEPS // RING), NSTEPS):
        c = step_work(jnp.int32(r), r % RING, c)

    pltpu.sync_copy(pool_v, out_hbm.at[pl.ds(wid * BPW, BPW)])


def _mm_body(p_ref, w_ref, o_ref):
    o_ref[:] = lax.dot_general(
        p_ref[:], w_ref[:],
        (((1,), (1,)), ((), ())),
        preferred_element_type=jnp.float32,
    )


def kernel(x, table, W_out):
    idx = jnp.reshape(x.astype(jnp.int32), (NW * NSTEPS, IPS))
    pooled = _sc_pool(idx, table)
    out = pl.pallas_call(
        _mm_body,
        out_shape=jax.ShapeDtypeStruct((B, N_OUT), jnp.float32),
    )(pooled, W_out)
    return out
